# baseline (speedup 1.0000x reference)
import functools

import jax
import jax.numpy as jnp
from jax import lax
from jax.experimental import pallas as pl
from jax.experimental.pallas import tpu as pltpu
from jax.experimental.pallas import tpu_sc as plsc

E = 8
D = 768
FF = 4 * 768
T = 2048
TS = 2 * T
TM = 128
NT = TS // TM
NG = NT + E - 1
FB = 768
NF = FF // FB
NC = 2
NS = 16
NW = NC * NS
TPW = T // NW
CH = 16


def _router_body(x_ref, wr_ref, p0_ref, p1_ref, x0_ref, x1_ref, offs_ref):
    xx = x_ref[...]
    logits = lax.dot_general(xx, wr_ref[...], (((1,), (1,)), ((), ())),
                             preferred_element_type=jnp.float32)
    mx = jnp.max(logits, axis=-1, keepdims=True)
    p = jnp.exp(logits - mx)
    p = p / jnp.sum(p, axis=-1, keepdims=True)
    lane = lax.broadcasted_iota(jnp.int32, (T, E), 1)
    p1 = jnp.max(p, axis=-1, keepdims=True)
    a1 = jnp.min(jnp.where(p == p1, lane, E), axis=-1, keepdims=True)
    mask1 = lane == a1
    pm = jnp.where(mask1, -jnp.inf, p)
    p2 = jnp.max(pm, axis=-1, keepdims=True)
    a2 = jnp.min(jnp.where(pm == p2, lane, E), axis=-1, keepdims=True)
    mask2 = lane == a2
    denom = jnp.maximum(p1 + p2, 1e-9)
    mf = jnp.where(mask1 | mask2, 1.0, 0.0)
    r0 = lax.broadcasted_iota(jnp.int32, (T, T), 0)
    c0 = lax.broadcasted_iota(jnp.int32, (T, T), 1)
    ltri = jnp.where(r0 > c0, 1.0, 0.0)
    ranks = lax.dot_general(ltri, mf, (((1,), (0,)), ((), ())),
                            preferred_element_type=jnp.float32)
    counts = jnp.sum(mf, axis=0, keepdims=True)
    re = lax.broadcasted_iota(jnp.int32, (E, E), 0)
    ce = lax.broadcasted_iota(jnp.int32, (E, E), 1)
    utri = jnp.where(re < ce, 1.0, 0.0)
    offs_f = lax.dot_general(counts, utri, (((1,), (0,)), ((), ())),
                             preferred_element_type=jnp.float32,
                             precision=lax.Precision.HIGHEST)
    pe = offs_f + ranks
    p0_ref[...] = jnp.round(jnp.sum(jnp.where(mask1, pe, 0.0),
                                    axis=1, keepdims=True)).astype(jnp.int32)
    p1_ref[...] = jnp.round(jnp.sum(jnp.where(mask2, pe, 0.0),
                                    axis=1, keepdims=True)).astype(jnp.int32)
    x0_ref[...] = jnp.sqrt(p1 / denom) * xx
    x1_ref[...] = jnp.sqrt(p2 / denom) * xx
    offs_ref[...] = jnp.round(offs_f).astype(jnp.int32)


def _router_call(flat, W_router):
    return pl.pallas_call(
        _router_body,
        grid=(1,),
        in_specs=[
            pl.BlockSpec((T, D), lambda g: (0, 0)),
            pl.BlockSpec((E, D), lambda g: (0, 0)),
        ],
        out_specs=(
            pl.BlockSpec((T, 1), lambda g: (0, 0)),
            pl.BlockSpec((T, 1), lambda g: (0, 0)),
            pl.BlockSpec((T, D), lambda g: (0, 0)),
            pl.BlockSpec((T, D), lambda g: (0, 0)),
            pl.BlockSpec((1, E), lambda g: (0, 0)),
        ),
        out_shape=(
            jax.ShapeDtypeStruct((T, 1), jnp.int32),
            jax.ShapeDtypeStruct((T, 1), jnp.int32),
            jax.ShapeDtypeStruct((T, D), jnp.float32),
            jax.ShapeDtypeStruct((T, D), jnp.float32),
            jax.ShapeDtypeStruct((1, E), jnp.int32),
        ),
    )(flat, W_router)


def _shared_body(x_ref, wsfc_ref, wsproj_ref, sh_ref):
    f = pl.program_id(0)
    hs = lax.dot_general(x_ref[...], wsfc_ref[...], (((1,), (1,)), ((), ())),
                         preferred_element_type=jnp.float32)
    a_s = jnp.square(jnp.maximum(hs, 0.0))
    ys = lax.dot_general(a_s, wsproj_ref[...], (((1,), (1,)), ((), ())),
                         preferred_element_type=jnp.float32)

    @pl.when(f == 0)
    def _():
        sh_ref[...] = ys

    @pl.when(f != 0)
    def _():
        sh_ref[...] += ys


def _shared_call(flat, Wsfc, Wsproj):
    return pl.pallas_call(
        _shared_body,
        grid=(NF,),
        in_specs=[
            pl.BlockSpec((T, D), lambda f: (0, 0)),
            pl.BlockSpec((FB, D), lambda f: (f, 0)),
            pl.BlockSpec((D, FB), lambda f: (0, f)),
        ],
        out_specs=pl.BlockSpec((T, D), lambda f: (0, 0)),
        out_shape=jax.ShapeDtypeStruct((T, D), jnp.float32),
    )(flat, Wsfc, Wsproj)


def _gmm_body(sm_ref, se_ref, first_ref, valid_ref, lo_ref, hi_ref,
              xs_ref, wfc_ref, wproj_ref, ys_ref):
    g = pl.program_id(0)

    @pl.when(valid_ref[g] == 1)
    def _():
        h = lax.dot_general(xs_ref[...], wfc_ref[0], (((1,), (1,)), ((), ())),
                            preferred_element_type=jnp.float32)
        a = jnp.square(jnp.maximum(h, 0.0))
        y = lax.dot_general(a, wproj_ref[0], (((1,), (1,)), ((), ())),
                            preferred_element_type=jnp.float32)
        rows = sm_ref[g] * TM + lax.broadcasted_iota(jnp.int32, (TM, 1), 0)
        msk = (rows >= lo_ref[g]) & (rows < hi_ref[g])
        ym = jnp.where(msk, y, 0.0)

        @pl.when(first_ref[g] == 1)
        def _w():
            ys_ref[...] = ym

        @pl.when(first_ref[g] == 0)
        def _acc():
            ys_ref[...] += ym


def _gmm_call(sm, se, first, valid, lo, hi, xs, Wfc, Wproj):
    grid_spec = pltpu.PrefetchScalarGridSpec(
        num_scalar_prefetch=6,
        grid=(NG,),
        in_specs=[
            pl.BlockSpec((TM, D), lambda g, sm, *_: (sm[g], 0)),
            pl.BlockSpec((1, FF, D), lambda g, sm, se, *_: (se[g], 0, 0)),
            pl.BlockSpec((1, D, FF), lambda g, sm, se, *_: (se[g], 0, 0)),
        ],
        out_specs=pl.BlockSpec((TM, D), lambda g, sm, *_: (sm[g], 0)),
    )
    return pl.pallas_call(
        _gmm_body,
        grid_spec=grid_spec,
        out_shape=jax.ShapeDtypeStruct((TS, D), jnp.float32),
    )(sm, se, first, valid, lo, hi, xs, Wfc, Wproj)


_SC_MESH = dict(core_axis_name="c", subcore_axis_name="s")


def _dispatch_sc(p0, p1, x0, x1):
    @functools.partial(
        pl.kernel,
        out_type=jax.ShapeDtypeStruct((TS, D), jnp.float32),
        mesh=plsc.VectorSubcoreMesh(**_SC_MESH),
        scratch_types=[
            pltpu.VMEM((TPW,), jnp.int32),
            pltpu.VMEM((TPW,), jnp.int32),
            pltpu.VMEM((TPW, D), jnp.float32),
            pltpu.VMEM((TPW, D), jnp.float32),
            pltpu.SemaphoreType.DMA,
        ],
    )
    def body(p0_hbm, p1_hbm, x0_hbm, x1_hbm, xs_hbm,
             p0loc, p1loc, x0loc, x1loc, sem):
        wid = lax.axis_index("s") * NC + lax.axis_index("c")
        base = wid * TPW
        pltpu.sync_copy(p0_hbm.at[pl.ds(base, TPW)], p0loc)
        pltpu.sync_copy(p1_hbm.at[pl.ds(base, TPW)], p1loc)
        pltpu.sync_copy(x0_hbm.at[pl.ds(base, TPW)], x0loc)
        pltpu.sync_copy(x1_hbm.at[pl.ds(base, TPW)], x1loc)
        copies = []
        for k in range(TPW // 16):
            sl = pl.ds(16 * k, 16)
            copies.append(
                pltpu.async_copy(x0loc.at[sl], xs_hbm.at[p0loc[sl]], sem))
            copies.append(
                pltpu.async_copy(x1loc.at[sl], xs_hbm.at[p1loc[sl]], sem))
        for cp in copies:
            cp.wait()

    return body(p0, p1, x0, x1)


def _combine_sc(p0, p1, ys, sh):
    @functools.partial(
        pl.kernel,
        out_type=jax.ShapeDtypeStruct((T, D), jnp.float32),
        mesh=plsc.VectorSubcoreMesh(**_SC_MESH),
        scratch_types=[
            pltpu.VMEM((TPW,), jnp.int32),
            pltpu.VMEM((TPW,), jnp.int32),
            pltpu.VMEM((CH, D), jnp.float32),
            pltpu.VMEM((CH, D), jnp.float32),
            pltpu.VMEM((CH, D), jnp.float32),
            pltpu.VMEM((CH, D), jnp.float32),
            pltpu.SemaphoreType.DMA,
        ],
    )
    def body(p0_hbm, p1_hbm, ys_hbm, sh_hbm, out_hbm,
             p0loc, p1loc, g0, g1, shloc, outloc, sem):
        wid = lax.axis_index("s") * NC + lax.axis_index("c")
        base = wid * TPW
        pltpu.sync_copy(p0_hbm.at[pl.ds(base, TPW)], p0loc)
        pltpu.sync_copy(p1_hbm.at[pl.ds(base, TPW)], p1loc)

        def chunk_body(ch, carry):
            tok0 = base + ch * CH
            cp0 = pltpu.async_copy(ys_hbm.at[p0loc[pl.ds(ch * CH, CH)]], g0, sem)
            cp1 = pltpu.async_copy(ys_hbm.at[p1loc[pl.ds(ch * CH, CH)]], g1, sem)
            pltpu.sync_copy(sh_hbm.at[pl.ds(tok0, CH)], shloc)
            cp0.wait()
            cp1.wait()

            def row_body(j, c1):
                def col_body(v, c2):
                    sl = pl.ds(v * 16, 16)
                    outloc[j, sl] = g0[j, sl] + g1[j, sl] + shloc[j, sl]
                    return c2

                lax.fori_loop(0, D // 16, col_body, 0)
                return c1

            lax.fori_loop(0, CH, row_body, 0)
            pltpu.sync_copy(outloc, out_hbm.at[pl.ds(tok0, CH)])
            return carry

        lax.fori_loop(0, TPW // CH, chunk_body, 0)

    return body(p0, p1, ys, sh)


def _metadata(offs):
    ends = jnp.concatenate([offs[1:], jnp.array([TS], jnp.int32)])
    m_lo = jnp.arange(NT, dtype=jnp.int32) * TM
    inc = (offs[None, :] < (m_lo[:, None] + TM)) & (ends[None, :] > m_lo[:, None])
    flat = inc.reshape(-1)
    order = jnp.argsort(~flat, stable=True).astype(jnp.int32)
    sel = order[:NG]
    valid = flat[sel].astype(jnp.int32)
    sm = (sel // E).astype(jnp.int32)
    se = (sel % E).astype(jnp.int32)
    nv = jnp.sum(flat.astype(jnp.int32))
    sm_last = sm[nv - 1]
    se_last = se[nv - 1]
    prev = jnp.concatenate([jnp.array([-1], jnp.int32), sm[:-1]])
    first = ((sm != prev) & (valid == 1)).astype(jnp.int32)
    sm = jnp.where(valid == 1, sm, sm_last)
    se = jnp.where(valid == 1, se, se_last)
    lo = jnp.maximum(offs[se], sm * TM)
    hi = jnp.minimum(ends[se], sm * TM + TM)
    return sm, se, first, valid, lo, hi


def kernel(x, W_router, Wfc, Wproj, Wsfc, Wsproj):
    B, S, Dd = x.shape
    flat = x.reshape(T, D)
    p0c, p1c, x0, x1, offs2 = _router_call(flat, W_router)
    p0 = p0c.reshape(TS // 2)
    p1 = p1c.reshape(TS // 2)
    xs = _dispatch_sc(p0, p1, x0, x1)
    sh = _shared_call(flat, Wsfc, Wsproj)
    sm, se, first, valid, lo, hi = _metadata(offs2[0])
    ys = _gmm_call(sm, se, first, valid, lo, hi, xs, Wfc, Wproj)
    out = _combine_sc(p0, p1, ys, sh)
    return out.reshape(B, S, Dd)


# speedup vs baseline: 1.2695x; 1.2695x over previous
import functools

import jax
import jax.numpy as jnp
from jax import lax
from jax.experimental import pallas as pl
from jax.experimental.pallas import tpu as pltpu
from jax.experimental.pallas import tpu_sc as plsc

E = 8
D = 768
FF = 4 * 768
T = 2048
TS = 2 * T
TM = 256
NT = TS // TM
NG = NT + E - 1
FB = 768
NF = FF // FB
NC = 2
NS = 16
NW = NC * NS
TPW = T // NW
CH = 16


def _router_body(x_ref, wr_ref, p0_ref, p1_ref, x0_ref, x1_ref, offs_ref):
    xx = x_ref[...]
    logits = lax.dot_general(xx, wr_ref[...], (((1,), (1,)), ((), ())),
                             preferred_element_type=jnp.float32)
    mx = jnp.max(logits, axis=-1, keepdims=True)
    p = jnp.exp(logits - mx)
    p = p / jnp.sum(p, axis=-1, keepdims=True)
    lane = lax.broadcasted_iota(jnp.int32, (T, E), 1)
    p1 = jnp.max(p, axis=-1, keepdims=True)
    a1 = jnp.min(jnp.where(p == p1, lane, E), axis=-1, keepdims=True)
    mask1 = lane == a1
    pm = jnp.where(mask1, -jnp.inf, p)
    p2 = jnp.max(pm, axis=-1, keepdims=True)
    a2 = jnp.min(jnp.where(pm == p2, lane, E), axis=-1, keepdims=True)
    mask2 = lane == a2
    denom = jnp.maximum(p1 + p2, 1e-9)
    mf = jnp.where(mask1 | mask2, 1.0, 0.0)
    r0 = lax.broadcasted_iota(jnp.int32, (T, T), 0)
    c0 = lax.broadcasted_iota(jnp.int32, (T, T), 1)
    ltri = jnp.where(r0 > c0, 1.0, 0.0)
    ranks = lax.dot_general(ltri, mf, (((1,), (0,)), ((), ())),
                            preferred_element_type=jnp.float32)
    counts = jnp.sum(mf, axis=0, keepdims=True)
    re = lax.broadcasted_iota(jnp.int32, (E, E), 0)
    ce = lax.broadcasted_iota(jnp.int32, (E, E), 1)
    utri = jnp.where(re < ce, 1.0, 0.0)
    offs_f = lax.dot_general(counts, utri, (((1,), (0,)), ((), ())),
                             preferred_element_type=jnp.float32,
                             precision=lax.Precision.HIGHEST)
    pe = offs_f + ranks
    p0_ref[...] = jnp.round(jnp.sum(jnp.where(mask1, pe, 0.0),
                                    axis=1, keepdims=True)).astype(jnp.int32)
    p1_ref[...] = jnp.round(jnp.sum(jnp.where(mask2, pe, 0.0),
                                    axis=1, keepdims=True)).astype(jnp.int32)
    x0_ref[...] = jnp.sqrt(p1 / denom) * xx
    x1_ref[...] = jnp.sqrt(p2 / denom) * xx
    offs_ref[...] = jnp.round(offs_f).astype(jnp.int32)


def _router_call(flat, W_router):
    return pl.pallas_call(
        _router_body,
        grid=(1,),
        in_specs=[
            pl.BlockSpec((T, D), lambda g: (0, 0)),
            pl.BlockSpec((E, D), lambda g: (0, 0)),
        ],
        out_specs=(
            pl.BlockSpec((T, 1), lambda g: (0, 0)),
            pl.BlockSpec((T, 1), lambda g: (0, 0)),
            pl.BlockSpec((T, D), lambda g: (0, 0)),
            pl.BlockSpec((T, D), lambda g: (0, 0)),
            pl.BlockSpec((1, E), lambda g: (0, 0)),
        ),
        out_shape=(
            jax.ShapeDtypeStruct((T, 1), jnp.int32),
            jax.ShapeDtypeStruct((T, 1), jnp.int32),
            jax.ShapeDtypeStruct((T, D), jnp.float32),
            jax.ShapeDtypeStruct((T, D), jnp.float32),
            jax.ShapeDtypeStruct((1, E), jnp.int32),
        ),
    )(flat, W_router)


def _shared_body(x_ref, wsfc_ref, wsproj_ref, sh_ref):
    f = pl.program_id(0)
    hs = lax.dot_general(x_ref[...], wsfc_ref[...], (((1,), (1,)), ((), ())),
                         preferred_element_type=jnp.float32)
    a_s = jnp.square(jnp.maximum(hs, 0.0))
    ys = lax.dot_general(a_s, wsproj_ref[...], (((1,), (1,)), ((), ())),
                         preferred_element_type=jnp.float32)

    @pl.when(f == 0)
    def _():
        sh_ref[...] = ys

    @pl.when(f != 0)
    def _():
        sh_ref[...] += ys


def _shared_call(flat, Wsfc, Wsproj):
    return pl.pallas_call(
        _shared_body,
        grid=(NF,),
        in_specs=[
            pl.BlockSpec((T, D), lambda f: (0, 0)),
            pl.BlockSpec((FB, D), lambda f: (f, 0)),
            pl.BlockSpec((D, FB), lambda f: (0, f)),
        ],
        out_specs=pl.BlockSpec((T, D), lambda f: (0, 0)),
        out_shape=jax.ShapeDtypeStruct((T, D), jnp.float32),
    )(flat, Wsfc, Wsproj)


def _gmm_body(sm_ref, se_ref, first_ref, valid_ref, lo_ref, hi_ref,
              xs_ref, wfc_ref, wproj_ref, ys_ref):
    g = pl.program_id(0)

    @pl.when(valid_ref[g] == 1)
    def _():
        h = lax.dot_general(xs_ref[...], wfc_ref[0], (((1,), (1,)), ((), ())),
                            preferred_element_type=jnp.float32)
        a = jnp.square(jnp.maximum(h, 0.0))
        y = lax.dot_general(a, wproj_ref[0], (((1,), (1,)), ((), ())),
                            preferred_element_type=jnp.float32)
        rows = sm_ref[g] * TM + lax.broadcasted_iota(jnp.int32, (TM, 1), 0)
        msk = (rows >= lo_ref[g]) & (rows < hi_ref[g])
        ym = jnp.where(msk, y, 0.0)

        @pl.when(first_ref[g] == 1)
        def _w():
            ys_ref[...] = ym

        @pl.when(first_ref[g] == 0)
        def _acc():
            ys_ref[...] += ym


def _gmm_call(sm, se, first, valid, lo, hi, xs, Wfc, Wproj):
    grid_spec = pltpu.PrefetchScalarGridSpec(
        num_scalar_prefetch=6,
        grid=(NG,),
        in_specs=[
            pl.BlockSpec((TM, D), lambda g, sm, *_: (sm[g], 0)),
            pl.BlockSpec((1, FF, D), lambda g, sm, se, *_: (se[g], 0, 0)),
            pl.BlockSpec((1, D, FF), lambda g, sm, se, *_: (se[g], 0, 0)),
        ],
        out_specs=pl.BlockSpec((TM, D), lambda g, sm, *_: (sm[g], 0)),
    )
    return pl.pallas_call(
        _gmm_body,
        grid_spec=grid_spec,
        out_shape=jax.ShapeDtypeStruct((TS, D), jnp.float32),
    )(sm, se, first, valid, lo, hi, xs, Wfc, Wproj)


_SC_MESH = dict(core_axis_name="c", subcore_axis_name="s")


def _dispatch_sc(p0, p1, x0, x1):
    @functools.partial(
        pl.kernel,
        out_type=jax.ShapeDtypeStruct((TS, D), jnp.float32),
        mesh=plsc.VectorSubcoreMesh(**_SC_MESH),
        scratch_types=[
            pltpu.VMEM((TPW,), jnp.int32),
            pltpu.VMEM((TPW,), jnp.int32),
            pltpu.VMEM((TPW, D), jnp.float32),
            pltpu.VMEM((TPW, D), jnp.float32),
            pltpu.SemaphoreType.DMA,
        ],
    )
    def body(p0_hbm, p1_hbm, x0_hbm, x1_hbm, xs_hbm,
             p0loc, p1loc, x0loc, x1loc, sem):
        wid = lax.axis_index("s") * NC + lax.axis_index("c")
        base = wid * TPW
        pltpu.sync_copy(p0_hbm.at[pl.ds(base, TPW)], p0loc)
        pltpu.sync_copy(p1_hbm.at[pl.ds(base, TPW)], p1loc)
        pltpu.sync_copy(x0_hbm.at[pl.ds(base, TPW)], x0loc)
        pltpu.sync_copy(x1_hbm.at[pl.ds(base, TPW)], x1loc)
        copies = []
        for k in range(TPW // 16):
            sl = pl.ds(16 * k, 16)
            copies.append(
                pltpu.async_copy(x0loc.at[sl], xs_hbm.at[p0loc[sl]], sem))
            copies.append(
                pltpu.async_copy(x1loc.at[sl], xs_hbm.at[p1loc[sl]], sem))
        for cp in copies:
            cp.wait()

    return body(p0, p1, x0, x1)


def _combine_sc(p0, p1, ys, sh):
    @functools.partial(
        pl.kernel,
        out_type=jax.ShapeDtypeStruct((T, D), jnp.float32),
        mesh=plsc.VectorSubcoreMesh(**_SC_MESH),
        scratch_types=[
            pltpu.VMEM((TPW,), jnp.int32),
            pltpu.VMEM((TPW,), jnp.int32),
            pltpu.VMEM((CH, D), jnp.float32),
            pltpu.VMEM((CH, D), jnp.float32),
            pltpu.VMEM((CH, D), jnp.float32),
            pltpu.VMEM((CH, D), jnp.float32),
            pltpu.SemaphoreType.DMA,
        ],
    )
    def body(p0_hbm, p1_hbm, ys_hbm, sh_hbm, out_hbm,
             p0loc, p1loc, g0, g1, shloc, outloc, sem):
        wid = lax.axis_index("s") * NC + lax.axis_index("c")
        base = wid * TPW
        pltpu.sync_copy(p0_hbm.at[pl.ds(base, TPW)], p0loc)
        pltpu.sync_copy(p1_hbm.at[pl.ds(base, TPW)], p1loc)

        def chunk_body(ch, carry):
            tok0 = base + ch * CH
            cp0 = pltpu.async_copy(ys_hbm.at[p0loc[pl.ds(ch * CH, CH)]], g0, sem)
            cp1 = pltpu.async_copy(ys_hbm.at[p1loc[pl.ds(ch * CH, CH)]], g1, sem)
            pltpu.sync_copy(sh_hbm.at[pl.ds(tok0, CH)], shloc)
            cp0.wait()
            cp1.wait()

            def row_body(j, c1):
                def col_body(v, c2):
                    sl = pl.ds(v * 16, 16)
                    outloc[j, sl] = g0[j, sl] + g1[j, sl] + shloc[j, sl]
                    return c2

                lax.fori_loop(0, D // 16, col_body, 0)
                return c1

            lax.fori_loop(0, CH, row_body, 0)
            pltpu.sync_copy(outloc, out_hbm.at[pl.ds(tok0, CH)])
            return carry

        lax.fori_loop(0, TPW // CH, chunk_body, 0)

    return body(p0, p1, ys, sh)


def _metadata(offs):
    ends = jnp.concatenate([offs[1:], jnp.array([TS], jnp.int32)])
    m_lo = jnp.arange(NT, dtype=jnp.int32) * TM
    inc = (offs[None, :] < (m_lo[:, None] + TM)) & (ends[None, :] > m_lo[:, None])
    flat = inc.reshape(-1)
    order = jnp.argsort(~flat, stable=True).astype(jnp.int32)
    sel = order[:NG]
    valid = flat[sel].astype(jnp.int32)
    sm = (sel // E).astype(jnp.int32)
    se = (sel % E).astype(jnp.int32)
    nv = jnp.sum(flat.astype(jnp.int32))
    sm_last = sm[nv - 1]
    se_last = se[nv - 1]
    prev = jnp.concatenate([jnp.array([-1], jnp.int32), sm[:-1]])
    first = ((sm != prev) & (valid == 1)).astype(jnp.int32)
    sm = jnp.where(valid == 1, sm, sm_last)
    se = jnp.where(valid == 1, se, se_last)
    lo = jnp.maximum(offs[se], sm * TM)
    hi = jnp.minimum(ends[se], sm * TM + TM)
    return sm, se, first, valid, lo, hi


def kernel(x, W_router, Wfc, Wproj, Wsfc, Wsproj):
    B, S, Dd = x.shape
    flat = x.reshape(T, D)
    p0c, p1c, x0, x1, offs2 = _router_call(flat, W_router)
    p0 = p0c.reshape(TS // 2)
    p1 = p1c.reshape(TS // 2)
    xs = _dispatch_sc(p0, p1, x0, x1)
    sh = _shared_call(flat, Wsfc, Wsproj)
    sm, se, first, valid, lo, hi = _metadata(offs2[0])
    ys = _gmm_call(sm, se, first, valid, lo, hi, xs, Wfc, Wproj)
    out = _combine_sc(p0, p1, ys, sh)
    return out.reshape(B, S, Dd)
